# baseline (speedup 1.0000x reference)
import jax
import jax.numpy as jnp
from jax.experimental import pallas as pl
from jax.experimental.pallas import tpu as pltpu
from jax.experimental.pallas import tpu_sc as plsc

D_MODEL = 1024
PAIR = 2
VOCAB = 2 ** PAIR
D_EFF = D_MODEL * PAIR
NUM_TILES = 32
CHUNK = 8
NBUF = 4


def _sc_gather(table, idx2d, n):
    per_tile = n // NUM_TILES
    nchunk = per_tile // CHUNK
    mesh = plsc.VectorSubcoreMesh(core_axis_name="c", subcore_axis_name="s")

    @pl.kernel(
        out_type=jax.ShapeDtypeStruct((n, D_EFF), table.dtype),
        mesh=mesh,
        scratch_types=[
            pltpu.VMEM((1, per_tile), jnp.int32),
        ]
        + [pltpu.VMEM((CHUNK, D_EFF), jnp.float32) for _ in range(NBUF)]
        + [pltpu.SemaphoreType.DMA for _ in range(2 * NBUF)],
    )
    def gather_kernel(table_hbm, idx_hbm, out_hbm, idx_v, *bufs_and_sems):
        bufs = bufs_and_sems[:NBUF]
        gsems = bufs_and_sems[NBUF : 2 * NBUF]
        wsems = bufs_and_sems[2 * NBUF : 3 * NBUF]

        c = jax.lax.axis_index("c")
        s = jax.lax.axis_index("s")
        t = c * 16 + s
        pltpu.async_copy(idx_hbm.at[pl.ds(t, 1), :], idx_v, gsems[0]).wait()
        row0 = t * per_tile

        def gather_copy(chunk, b):
            rows = idx_v.at[0, pl.ds(chunk * CHUNK, CHUNK)]
            return pltpu.make_async_copy(table_hbm.at[rows], bufs[b], gsems[b])

        def write_copy(chunk, b):
            dst = out_hbm.at[pl.ds(row0 + chunk * CHUNK, CHUNK), :]
            return pltpu.make_async_copy(bufs[b], dst, wsems[b])

        for b in range(NBUF):
            gather_copy(b, b).start()

        @pl.loop(0, nchunk - NBUF, step=NBUF)
        def _(j):
            for b in range(NBUF):
                gather_copy(j + b, b).wait()
                write_copy(j + b, b).start()
            for b in range(NBUF):
                write_copy(j + b, b).wait()
                gather_copy(j + NBUF + b, b).start()

        for b in range(NBUF):
            gather_copy(nchunk - NBUF + b, b).wait()
            write_copy(nchunk - NBUF + b, b).start()
        for b in range(NBUF):
            write_copy(nchunk - NBUF + b, b).wait()

    return gather_kernel(table, idx2d)


def kernel(mask_bits, table):
    b, s = mask_bits.shape
    n = (b * s) // PAIR
    bits = mask_bits.astype(jnp.int32).reshape(-1, PAIR)
    idx = (bits[:, 0] * 2 + bits[:, 1]).reshape(NUM_TILES, n // NUM_TILES)
    combo = jnp.stack(
        [
            jnp.concatenate([table[c // 2], table[c % 2]])
            for c in range(VOCAB)
        ]
    )
    reps = 16
    table_rep = jnp.tile(combo, (NUM_TILES * reps, 1))
    tile_off = jnp.arange(NUM_TILES, dtype=jnp.int32)[:, None] * reps
    pos_off = jnp.arange(n // NUM_TILES, dtype=jnp.int32)[None, :] % reps
    idx = idx + VOCAB * (tile_off + pos_off)
    out = _sc_gather(table_rep, idx, n)
    return out.reshape(b, s, D_MODEL)


# speedup vs baseline: 1.7372x; 1.7372x over previous
import jax
import jax.numpy as jnp
from jax.experimental import pallas as pl
from jax.experimental.pallas import tpu as pltpu
from jax.experimental.pallas import tpu_sc as plsc

D_MODEL = 1024
NUM_TILES = 32
CHUNK = 8
NBUF = 8


def _sc_gather(table, idx2d, n):
    per_tile = n // NUM_TILES
    nchunk = per_tile // CHUNK
    mesh = plsc.VectorSubcoreMesh(core_axis_name="c", subcore_axis_name="s")

    @pl.kernel(
        out_type=jax.ShapeDtypeStruct((n, D_MODEL), table.dtype),
        mesh=mesh,
        scratch_types=[
            pltpu.VMEM((1, per_tile), jnp.int32),
        ]
        + [pltpu.VMEM((CHUNK, D_MODEL), jnp.float32) for _ in range(NBUF)]
        + [pltpu.SemaphoreType.DMA for _ in range(2 * NBUF)],
    )
    def gather_kernel(table_hbm, idx_hbm, out_hbm, idx_v, *bufs_and_sems):
        bufs = bufs_and_sems[:NBUF]
        gsems = bufs_and_sems[NBUF : 2 * NBUF]
        wsems = bufs_and_sems[2 * NBUF : 3 * NBUF]

        c = jax.lax.axis_index("c")
        s = jax.lax.axis_index("s")
        t = c * 16 + s
        pltpu.async_copy(idx_hbm.at[pl.ds(t, 1), :], idx_v, gsems[0]).wait()
        row0 = t * per_tile

        def gather_copy(chunk, b):
            rows = idx_v.at[0, pl.ds(chunk * CHUNK, CHUNK)]
            return pltpu.make_async_copy(table_hbm.at[rows], bufs[b], gsems[b])

        def write_copy(chunk, b):
            dst = out_hbm.at[pl.ds(row0 + chunk * CHUNK, CHUNK), :]
            return pltpu.make_async_copy(bufs[b], dst, wsems[b])

        for b in range(NBUF):
            gather_copy(b, b).start()

        @pl.loop(0, nchunk - NBUF, step=NBUF)
        def _(j):
            for b in range(NBUF):
                gather_copy(j + b, b).wait()
                write_copy(j + b, b).start()
            for b in range(NBUF):
                write_copy(j + b, b).wait()
                gather_copy(j + NBUF + b, b).start()

        for b in range(NBUF):
            gather_copy(nchunk - NBUF + b, b).wait()
            write_copy(nchunk - NBUF + b, b).start()
        for b in range(NBUF):
            write_copy(nchunk - NBUF + b, b).wait()

    return gather_kernel(table, idx2d)


TC_BLK = 512
SC_ROWS = 8192


def _tc_select(bits3d, table, nblk):
    def body(bits_ref, table_ref, out_ref):
        bb = bits_ref[0, 0, :]
        out_ref[0] = jnp.where(
            bb[:, None] == 0, table_ref[0:1, :], table_ref[1:2, :]
        )

    return pl.pallas_call(
        body,
        grid=(nblk,),
        in_specs=[
            pl.BlockSpec((1, 1, TC_BLK), lambda i: (i, 0, 0)),
            pl.BlockSpec((2, D_MODEL), lambda i: (0, 0)),
        ],
        out_specs=pl.BlockSpec((1, TC_BLK, D_MODEL), lambda i: (i, 0, 0)),
        out_shape=jax.ShapeDtypeStruct((nblk, TC_BLK, D_MODEL), jnp.float32),
    )(bits3d, table)


def kernel(mask_bits, table):
    b, s = mask_bits.shape
    n = b * s
    flat = mask_bits.astype(jnp.int32).reshape(-1)
    n_tc = n - SC_ROWS
    nblk = n_tc // TC_BLK

    out_tc = _tc_select(flat[:n_tc].reshape(nblk, 1, TC_BLK), table, nblk)
    out_tc = out_tc.reshape(n_tc, D_MODEL)

    idx = flat[n_tc:].reshape(NUM_TILES, SC_ROWS // NUM_TILES)
    reps = 16
    table_rep = jnp.tile(table, (NUM_TILES * reps, 1))
    tile_off = jnp.arange(NUM_TILES, dtype=jnp.int32)[:, None] * reps
    pos_off = jnp.arange(SC_ROWS // NUM_TILES, dtype=jnp.int32)[None, :] % reps
    idx = idx + 2 * (tile_off + pos_off)
    out_sc = _sc_gather(table_rep, idx, SC_ROWS)

    out = jnp.concatenate([out_tc, out_sc], axis=0)
    return out.reshape(b, s, D_MODEL)


# speedup vs baseline: 3.1959x; 1.8397x over previous
import jax
import jax.numpy as jnp
from jax.experimental import pallas as pl
from jax.experimental.pallas import tpu as pltpu
from jax.experimental.pallas import tpu_sc as plsc

D_MODEL = 1024
NUM_TILES = 32
CHUNK = 8
NBUF = 8


def _sc_gather(table, idx2d, n_total, n_sc):
    per_tile = n_sc // NUM_TILES
    nchunk = per_tile // CHUNK
    base_row = n_total - n_sc
    mesh = plsc.VectorSubcoreMesh(core_axis_name="c", subcore_axis_name="s")

    @pl.kernel(
        out_type=jax.ShapeDtypeStruct((n_total, D_MODEL), table.dtype),
        mesh=mesh,
        scratch_types=[
            pltpu.VMEM((1, per_tile), jnp.int32),
        ]
        + [pltpu.VMEM((CHUNK, D_MODEL), jnp.float32) for _ in range(NBUF)]
        + [pltpu.SemaphoreType.DMA for _ in range(2 * NBUF)],
    )
    def gather_kernel(table_hbm, idx_hbm, out_hbm, idx_v, *bufs_and_sems):
        bufs = bufs_and_sems[:NBUF]
        gsems = bufs_and_sems[NBUF : 2 * NBUF]
        wsems = bufs_and_sems[2 * NBUF : 3 * NBUF]

        c = jax.lax.axis_index("c")
        s = jax.lax.axis_index("s")
        t = c * 16 + s
        pltpu.async_copy(idx_hbm.at[pl.ds(t, 1), :], idx_v, gsems[0]).wait()
        row0 = base_row + t * per_tile

        def gather_copy(chunk, b):
            rows = idx_v.at[0, pl.ds(chunk * CHUNK, CHUNK)]
            return pltpu.make_async_copy(table_hbm.at[rows], bufs[b], gsems[b])

        def write_copy(chunk, b):
            dst = out_hbm.at[pl.ds(row0 + chunk * CHUNK, CHUNK), :]
            return pltpu.make_async_copy(bufs[b], dst, wsems[b])

        for b in range(NBUF):
            gather_copy(b, b).start()

        @pl.loop(0, nchunk - NBUF, step=NBUF)
        def _(j):
            for b in range(NBUF):
                gather_copy(j + b, b).wait()
                write_copy(j + b, b).start()
            for b in range(NBUF):
                write_copy(j + b, b).wait()
                gather_copy(j + NBUF + b, b).start()

        for b in range(NBUF):
            gather_copy(nchunk - NBUF + b, b).wait()
            write_copy(nchunk - NBUF + b, b).start()
        for b in range(NBUF):
            write_copy(nchunk - NBUF + b, b).wait()

    return gather_kernel(table, idx2d)


TC_BLK = 512
SC_ROWS = 8192


def _tc_select(sc_out3d, bits3d, table, nblk_head):
    def body(full_ref, bits_ref, table_ref, out_ref):
        del full_ref
        bb = bits_ref[0, 0, :]
        out_ref[0] = jnp.where(
            bb[:, None] == 0, table_ref[0:1, :], table_ref[1:2, :]
        )

    nblk_total = sc_out3d.shape[0]
    return pl.pallas_call(
        body,
        grid=(nblk_head,),
        in_specs=[
            pl.BlockSpec(memory_space=pl.ANY),
            pl.BlockSpec((1, 1, TC_BLK), lambda i: (i, 0, 0)),
            pl.BlockSpec((2, D_MODEL), lambda i: (0, 0)),
        ],
        out_specs=pl.BlockSpec((1, TC_BLK, D_MODEL), lambda i: (i, 0, 0)),
        out_shape=jax.ShapeDtypeStruct(
            (nblk_total, TC_BLK, D_MODEL), jnp.float32
        ),
        input_output_aliases={0: 0},
    )(sc_out3d, bits3d, table)


def kernel(mask_bits, table):
    b, s = mask_bits.shape
    n = b * s
    flat = mask_bits.astype(jnp.int32).reshape(-1)
    n_tc = n - SC_ROWS
    nblk_head = n_tc // TC_BLK

    idx = flat[n_tc:].reshape(NUM_TILES, SC_ROWS // NUM_TILES)
    reps = 16
    table_rep = jnp.tile(table, (NUM_TILES * reps, 1))
    tile_off = jnp.arange(NUM_TILES, dtype=jnp.int32)[:, None] * reps
    pos_off = jnp.arange(SC_ROWS // NUM_TILES, dtype=jnp.int32)[None, :] % reps
    idx = idx + 2 * (tile_off + pos_off)
    out_sc = _sc_gather(table_rep, idx, n, SC_ROWS)

    out = _tc_select(
        out_sc.reshape(n // TC_BLK, TC_BLK, D_MODEL),
        flat[:n_tc].reshape(nblk_head, 1, TC_BLK),
        table,
        nblk_head,
    )
    return out.reshape(b, s, D_MODEL)


# speedup vs baseline: 3.4156x; 1.0688x over previous
import jax
import jax.numpy as jnp
from jax.experimental import pallas as pl
from jax.experimental.pallas import tpu as pltpu
from jax.experimental.pallas import tpu_sc as plsc

D_MODEL = 1024
NUM_TILES = 32
CHUNK = 8
NBUF = 8


def _sc_gather(table, idx2d, n_total, n_sc):
    per_tile = n_sc // NUM_TILES
    nchunk = per_tile // CHUNK
    base_row = n_total - n_sc
    mesh = plsc.VectorSubcoreMesh(core_axis_name="c", subcore_axis_name="s")

    @pl.kernel(
        out_type=jax.ShapeDtypeStruct((n_total, D_MODEL), table.dtype),
        mesh=mesh,
        scratch_types=[
            pltpu.VMEM((1, per_tile), jnp.int32),
        ]
        + [pltpu.VMEM((CHUNK, D_MODEL), jnp.float32) for _ in range(NBUF)]
        + [pltpu.SemaphoreType.DMA for _ in range(2 * NBUF)],
    )
    def gather_kernel(table_hbm, idx_hbm, out_hbm, idx_v, *bufs_and_sems):
        bufs = bufs_and_sems[:NBUF]
        gsems = bufs_and_sems[NBUF : 2 * NBUF]
        wsems = bufs_and_sems[2 * NBUF : 3 * NBUF]

        c = jax.lax.axis_index("c")
        s = jax.lax.axis_index("s")
        t = c * 16 + s
        pltpu.async_copy(idx_hbm.at[pl.ds(t, 1), :], idx_v, gsems[0]).wait()
        row0 = base_row + t * per_tile

        def gather_copy(chunk, b):
            rows = idx_v.at[0, pl.ds(chunk * CHUNK, CHUNK)]
            return pltpu.make_async_copy(table_hbm.at[rows], bufs[b], gsems[b])

        def write_copy(chunk, b):
            dst = out_hbm.at[pl.ds(row0 + chunk * CHUNK, CHUNK), :]
            return pltpu.make_async_copy(bufs[b], dst, wsems[b])

        for b in range(NBUF):
            gather_copy(b, b).start()

        @pl.loop(0, nchunk - NBUF, step=NBUF)
        def _(j):
            for b in range(NBUF):
                gather_copy(j + b, b).wait()
                write_copy(j + b, b).start()
            for b in range(NBUF):
                write_copy(j + b, b).wait()
                gather_copy(j + NBUF + b, b).start()

        for b in range(NBUF):
            gather_copy(nchunk - NBUF + b, b).wait()
            write_copy(nchunk - NBUF + b, b).start()
        for b in range(NBUF):
            write_copy(nchunk - NBUF + b, b).wait()

    return gather_kernel(table, idx2d)


TC_BLK = 512
SC_ROWS = 4096


def _tc_select(sc_out3d, bits3d, table, nblk_head):
    def body(full_ref, bits_ref, table_ref, out_ref):
        del full_ref
        bb = bits_ref[0, 0, :]
        out_ref[0] = jnp.where(
            bb[:, None] == 0, table_ref[0:1, :], table_ref[1:2, :]
        )

    nblk_total = sc_out3d.shape[0]
    return pl.pallas_call(
        body,
        grid=(nblk_head,),
        in_specs=[
            pl.BlockSpec(memory_space=pl.ANY),
            pl.BlockSpec((1, 1, TC_BLK), lambda i: (i, 0, 0)),
            pl.BlockSpec((2, D_MODEL), lambda i: (0, 0)),
        ],
        out_specs=pl.BlockSpec((1, TC_BLK, D_MODEL), lambda i: (i, 0, 0)),
        out_shape=jax.ShapeDtypeStruct(
            (nblk_total, TC_BLK, D_MODEL), jnp.float32
        ),
        input_output_aliases={0: 0},
    )(sc_out3d, bits3d, table)


def kernel(mask_bits, table):
    b, s = mask_bits.shape
    n = b * s
    flat = mask_bits.astype(jnp.int32).reshape(-1)
    n_tc = n - SC_ROWS
    nblk_head = n_tc // TC_BLK

    idx = flat[n_tc:].reshape(NUM_TILES, SC_ROWS // NUM_TILES)
    reps = 16
    table_rep = jnp.tile(table, (NUM_TILES * reps, 1))
    tile_off = jnp.arange(NUM_TILES, dtype=jnp.int32)[:, None] * reps
    pos_off = jnp.arange(SC_ROWS // NUM_TILES, dtype=jnp.int32)[None, :] % reps
    idx = idx + 2 * (tile_off + pos_off)
    out_sc = _sc_gather(table_rep, idx, n, SC_ROWS)

    out = _tc_select(
        out_sc.reshape(n // TC_BLK, TC_BLK, D_MODEL),
        flat[:n_tc].reshape(nblk_head, 1, TC_BLK),
        table,
        nblk_head,
    )
    return out.reshape(b, s, D_MODEL)


# speedup vs baseline: 3.9186x; 1.1472x over previous
import jax
import jax.numpy as jnp
from jax.experimental import pallas as pl
from jax.experimental.pallas import tpu as pltpu
from jax.experimental.pallas import tpu_sc as plsc

D_MODEL = 1024
NUM_TILES = 32
CHUNK = 8
NBUF = 8


def _sc_gather(table, idx2d, n_total, n_sc):
    per_tile = n_sc // NUM_TILES
    nchunk = per_tile // CHUNK
    base_row = n_total - n_sc
    mesh = plsc.VectorSubcoreMesh(core_axis_name="c", subcore_axis_name="s")

    @pl.kernel(
        out_type=jax.ShapeDtypeStruct((n_total, D_MODEL), table.dtype),
        mesh=mesh,
        scratch_types=[
            pltpu.VMEM((1, per_tile), jnp.int32),
        ]
        + [pltpu.VMEM((CHUNK, D_MODEL), jnp.float32) for _ in range(NBUF)]
        + [pltpu.SemaphoreType.DMA for _ in range(2 * NBUF)],
    )
    def gather_kernel(table_hbm, idx_hbm, out_hbm, idx_v, *bufs_and_sems):
        bufs = bufs_and_sems[:NBUF]
        gsems = bufs_and_sems[NBUF : 2 * NBUF]
        wsems = bufs_and_sems[2 * NBUF : 3 * NBUF]

        c = jax.lax.axis_index("c")
        s = jax.lax.axis_index("s")
        t = c * 16 + s
        pltpu.async_copy(idx_hbm.at[pl.ds(t, 1), :], idx_v, gsems[0]).wait()
        row0 = base_row + t * per_tile

        def gather_copy(chunk, b):
            rows = idx_v.at[0, pl.ds(chunk * CHUNK, CHUNK)]
            return pltpu.make_async_copy(table_hbm.at[rows], bufs[b], gsems[b])

        def write_copy(chunk, b):
            dst = out_hbm.at[pl.ds(row0 + chunk * CHUNK, CHUNK), :]
            return pltpu.make_async_copy(bufs[b], dst, wsems[b])

        for b in range(NBUF):
            gather_copy(b, b).start()

        @pl.loop(0, nchunk - NBUF, step=NBUF)
        def _(j):
            for b in range(NBUF):
                gather_copy(j + b, b).wait()
                write_copy(j + b, b).start()
            for b in range(NBUF):
                write_copy(j + b, b).wait()
                gather_copy(j + NBUF + b, b).start()

        for b in range(NBUF):
            gather_copy(nchunk - NBUF + b, b).wait()
            write_copy(nchunk - NBUF + b, b).start()
        for b in range(NBUF):
            write_copy(nchunk - NBUF + b, b).wait()

    return gather_kernel(table, idx2d)


TC_BLK = 2048
SC_ROWS = 4096


def _tc_select(sc_out3d, bits3d, table, nblk_head):
    def body(full_ref, bits_ref, table_ref, out_ref):
        del full_ref
        bb = bits_ref[0, 0, :]
        out_ref[0] = jnp.where(
            bb[:, None] == 0, table_ref[0:1, :], table_ref[1:2, :]
        )

    nblk_total = sc_out3d.shape[0]
    return pl.pallas_call(
        body,
        grid=(nblk_head,),
        in_specs=[
            pl.BlockSpec(memory_space=pl.ANY),
            pl.BlockSpec((1, 1, TC_BLK), lambda i: (i, 0, 0)),
            pl.BlockSpec((2, D_MODEL), lambda i: (0, 0)),
        ],
        out_specs=pl.BlockSpec((1, TC_BLK, D_MODEL), lambda i: (i, 0, 0)),
        out_shape=jax.ShapeDtypeStruct(
            (nblk_total, TC_BLK, D_MODEL), jnp.float32
        ),
        input_output_aliases={0: 0},
    )(sc_out3d, bits3d, table)


def kernel(mask_bits, table):
    b, s = mask_bits.shape
    n = b * s
    flat = mask_bits.astype(jnp.int32).reshape(-1)
    n_tc = n - SC_ROWS
    nblk_head = n_tc // TC_BLK

    idx = flat[n_tc:].reshape(NUM_TILES, SC_ROWS // NUM_TILES)
    reps = 16
    table_rep = jnp.tile(table, (NUM_TILES * reps, 1))
    tile_off = jnp.arange(NUM_TILES, dtype=jnp.int32)[:, None] * reps
    pos_off = jnp.arange(SC_ROWS // NUM_TILES, dtype=jnp.int32)[None, :] % reps
    idx = idx + 2 * (tile_off + pos_off)
    out_sc = _sc_gather(table_rep, idx, n, SC_ROWS)

    out = _tc_select(
        out_sc.reshape(n // TC_BLK, TC_BLK, D_MODEL),
        flat[:n_tc].reshape(nblk_head, 1, TC_BLK),
        table,
        nblk_head,
    )
    return out.reshape(b, s, D_MODEL)
